# baseline (speedup 1.0000x reference)
import functools

import jax
import jax.numpy as jnp
from jax import lax
from jax.experimental import pallas as pl
from jax.experimental.pallas import tpu as pltpu
from jax.experimental.pallas import tpu_sc as plsc

NN = 10000
EE = 320000
HH = 128
NC = 2
NS = 16
NW = NC * NS
NPAD = 10240
EPW = 10240
CH = 128
NCHUNK = EPW // CH
RPS = NPAD // NS
BS = 320
BPC = 96
TOTP = NW * BPC * CH
PARTN = TOTP + CH
PR = PARTN // CH
MAGIC = 52429
BLK = 1000
GRID = NN // BLK

_mesh = plsc.VectorSubcoreMesh(core_axis_name="c", subcore_axis_name="s")


def _cnt_body(packed, zeros1, degp_out, cnt_out, degsh, pk, dstv, onesv, cntv):
    c = lax.axis_index("c")
    s = lax.axis_index("s")
    wid = c * NS + s
    for b in range(32):
        cntv[pl.ds(16 * b, 16)] = jnp.zeros((16,), jnp.int32)
    for j in range(CH // 16):
        onesv[pl.ds(16 * j, 16)] = jnp.ones((16,), jnp.float32)
    pltpu.sync_copy(zeros1.at[pl.ds(s * RPS, RPS)], degsh.at[pl.ds(s * RPS, RPS)])
    pltpu.sync_copy(packed.at[wid], pk)
    plsc.subcore_barrier()

    def chunk(i, carry):
        for g in range(CH // 16):
            p = pk[i, pl.ds(16 * g, 16)]
            d = lax.bitwise_and(p, 16383)
            dstv[pl.ds(16 * g, 16)] = d
            valid = d < NN
            bk = lax.shift_right_logical(d * MAGIC, 24)
            for bb in range(32):
                m = valid & (bk == bb)
                cb = cntv[pl.ds(16 * bb, 16)]
                cntv[pl.ds(16 * bb, 16)] = cb + jnp.where(m, 1, 0)
        pltpu.sync_copy(onesv, degsh.at[dstv], add=True)
        return carry

    lax.fori_loop(0, NCHUNK, chunk, 0)
    plsc.subcore_barrier()
    pltpu.sync_copy(degsh.at[pl.ds(s * RPS, RPS)],
                    degp_out.at[c, pl.ds(s * RPS, RPS)])
    pltpu.sync_copy(cntv, cnt_out.at[wid])


def _cnt_call(packed, zeros1):
    kfn = pl.kernel(
        _cnt_body,
        out_type=[
            jax.ShapeDtypeStruct((NC, NPAD), jnp.float32),
            jax.ShapeDtypeStruct((NW, 512), jnp.int32),
        ],
        mesh=_mesh,
        scratch_types=[
            pltpu.VMEM_SHARED((NPAD,), jnp.float32),
            pltpu.VMEM((NCHUNK, CH), jnp.int32),
            pltpu.VMEM((CH,), jnp.int32),
            pltpu.VMEM((CH,), jnp.float32),
            pltpu.VMEM((512,), jnp.int32),
        ],
    )
    return kfn(packed, zeros1)


def _part_body(packed, base3, parted_out, pk, cntv,
               rec0, slot0, rec1, slot1, psem0, psem1):
    c = lax.axis_index("c")
    s = lax.axis_index("s")
    wid = c * NS + s
    pltpu.sync_copy(packed.at[wid], pk)
    pltpu.sync_copy(base3.at[wid], cntv)
    iota16 = lax.iota(jnp.int32, 16)

    def compute(i, recv, slotv):
        for g in range(CH // 16):
            p = pk[i, pl.ds(16 * g, 16)]
            src = lax.shift_right_logical(p, 14)
            d = lax.bitwise_and(p, 16383)
            valid = d < NN
            bk = lax.shift_right_logical(d * MAGIC, 24)
            local = d - bk * BS
            rec = lax.bitwise_or(lax.shift_left(src, 9), local)
            slot = TOTP + g * 16 + iota16
            for bb in range(32):
                m = valid & (bk == bb)
                cb = cntv[pl.ds(16 * bb, 16)]
                slot = jnp.where(m, cb, slot)
                cntv[pl.ds(16 * bb, 16)] = cb + jnp.where(m, 1, 0)
            recv[pl.ds(16 * g, 16)] = rec
            slotv[pl.ds(16 * g, 16)] = slot

    compute(0, rec0, slot0)
    pltpu.async_copy(rec0, parted_out.at[slot0], psem0)

    def body(k, carry):
        i0 = 2 * k
        compute(i0 + 1, rec1, slot1)
        pltpu.async_copy(rec1, parted_out.at[slot1], psem1)
        pltpu.make_async_copy(rec0, parted_out.at[slot0], psem0).wait()

        @pl.when(i0 + 2 < NCHUNK)
        def _():
            compute(i0 + 2, rec0, slot0)
            pltpu.async_copy(rec0, parted_out.at[slot0], psem0)

        pltpu.make_async_copy(rec1, parted_out.at[slot1], psem1).wait()
        return carry

    lax.fori_loop(0, NCHUNK // 2, body, 0)


def _part_call(packed, base3):
    kfn = pl.kernel(
        _part_body,
        out_type=jax.ShapeDtypeStruct((PARTN,), jnp.int32),
        mesh=_mesh,
        scratch_types=[
            pltpu.VMEM((NCHUNK, CH), jnp.int32),
            pltpu.VMEM((512,), jnp.int32),
            pltpu.VMEM((CH,), jnp.int32),
            pltpu.VMEM((CH,), jnp.int32),
            pltpu.VMEM((CH,), jnp.int32),
            pltpu.VMEM((CH,), jnp.int32),
            pltpu.SemaphoreType.DMA,
            pltpu.SemaphoreType.DMA,
        ],
    )
    return kfn(packed, base3)


def _apply_body(table, parted2, zeros2, blb, nchb, out,
                acc, pk3, s0v, l0v, s1v, l1v, rows0, rows1, blv, ncv,
                gsem0, gsem1):
    c = lax.axis_index("c")
    s = lax.axis_index("s")
    w = c * NS + s
    pltpu.sync_copy(zeros2.at[pl.ds(0, BS + 8)], acc)
    pltpu.sync_copy(parted2.at[pl.ds(w * BPC, BPC)], pk3)
    pltpu.sync_copy(blb.at[w], blv)
    pltpu.sync_copy(nchb.at[w], ncv)
    bl_vec = blv[pl.ds(0, 16)]
    nch = ncv[pl.ds(0, 16)][0]
    iota16 = lax.iota(jnp.int32, 16)

    def unpack(i, sv, lv):
        base_row = i * CH
        for g in range(CH // 16):
            p = pk3[i, pl.ds(16 * g, 16)]
            sv[pl.ds(16 * g, 16)] = jnp.minimum(
                lax.shift_right_logical(p, 9), NN - 1)
            rowpos = base_row + g * 16 + iota16
            lv[pl.ds(16 * g, 16)] = jnp.where(
                rowpos < bl_vec, lax.bitwise_and(p, 511), BS)

    def accum(rows, lv):
        for r in range(CH):
            if r % 16 == 0:
                vv = lv[pl.ds(r, 16)]
            d = vv[r % 16]
            for j in range(HH // 16):
                plsc.addupdate(acc.at[d, pl.ds(16 * j, 16)],
                               rows[r, pl.ds(16 * j, 16)])

    unpack(0, s0v, l0v)
    pltpu.async_copy(table.at[s0v], rows0, gsem0)
    unpack(1, s1v, l1v)
    pltpu.async_copy(table.at[s1v], rows1, gsem1)

    def body(m, carry):
        i0 = 2 * m
        pltpu.make_async_copy(table.at[s0v], rows0, gsem0).wait()
        accum(rows0, l0v)
        unpack(lax.rem(i0 + 2, nch), s0v, l0v)
        pltpu.async_copy(table.at[s0v], rows0, gsem0)
        pltpu.make_async_copy(table.at[s1v], rows1, gsem1).wait()
        accum(rows1, l1v)
        unpack(lax.rem(i0 + 3, nch), s1v, l1v)
        pltpu.async_copy(table.at[s1v], rows1, gsem1)
        return carry

    lax.fori_loop(0, nch // 2, body, 0)
    pltpu.make_async_copy(table.at[s0v], rows0, gsem0).wait()
    pltpu.make_async_copy(table.at[s1v], rows1, gsem1).wait()
    pltpu.sync_copy(acc.at[pl.ds(0, BS)], out.at[pl.ds(w * BS, BS)])


def _apply_call(table, parted2, zeros2, blb, nchb):
    kfn = pl.kernel(
        _apply_body,
        out_type=jax.ShapeDtypeStruct((NPAD, HH), jnp.float32),
        mesh=_mesh,
        scratch_types=[
            pltpu.VMEM((BS + 8, HH), jnp.float32),
            pltpu.VMEM((BPC, CH), jnp.int32),
            pltpu.VMEM((CH,), jnp.int32),
            pltpu.VMEM((CH,), jnp.int32),
            pltpu.VMEM((CH,), jnp.int32),
            pltpu.VMEM((CH,), jnp.int32),
            pltpu.VMEM((CH, HH), jnp.float32),
            pltpu.VMEM((CH, HH), jnp.float32),
            pltpu.VMEM((16,), jnp.int32),
            pltpu.VMEM((16,), jnp.int32),
            pltpu.SemaphoreType.DMA,
            pltpu.SemaphoreType.DMA,
        ],
    )
    return kfn(table, parted2, zeros2, blb, nchb)


def _pref_body(cntT, baseT_out, bl_out, nch_out):
    cf = cntT[...].astype(jnp.float32)
    r = lax.broadcasted_iota(jnp.int32, (512, 512), 0)
    c2 = lax.broadcasted_iota(jnp.int32, (512, 512), 1)
    tri = (r < c2).astype(jnp.float32)
    pre = jnp.dot(cf, tri, preferred_element_type=jnp.float32)
    boff = lax.broadcasted_iota(jnp.int32, (32, 512), 0) * (BPC * CH)
    baseT_out[...] = pre.astype(jnp.int32) + boff
    bl = jnp.sum(cntT[...], axis=1, keepdims=True)
    bl_out[...] = jnp.broadcast_to(bl, (32, 16))
    nch = jnp.minimum(((bl + 2 * CH - 1) >> 8) * 2, BPC)
    nch_out[...] = jnp.broadcast_to(nch, (32, 16))


def _pref_call(cntT):
    return pl.pallas_call(
        _pref_body,
        out_shape=[
            jax.ShapeDtypeStruct((32, 512), jnp.int32),
            jax.ShapeDtypeStruct((32, 16), jnp.int32),
            jax.ShapeDtypeStruct((32, 16), jnp.int32),
        ],
    )(cntT)


def _a0_body(xr, wr, degr, out):
    dinv = lax.rsqrt(degr[0] + degr[1] + 1.0)
    out[...] = jnp.dot(xr[...], wr[...],
                       preferred_element_type=jnp.float32) * dinv


def _a_body(yr, sr, qr, gr, br, wr, degr, out):
    dinv = lax.rsqrt(degr[0] + degr[1] + 1.0)
    mu = sr[...] * (1.0 / NN)
    var = qr[...] * (1.0 / NN) - mu * mu
    a = gr[...] * (yr[...] - mu) * lax.rsqrt(var + 1e-5) + br[...]
    a = jnp.maximum(a, 0.0)
    out[...] = jnp.dot(a, wr[...], preferred_element_type=jnp.float32) * dinv


def _b_body(pr, hr, br, degr, y_out, s_out, q_out, *, stats):
    dinv = lax.rsqrt(degr[0] + degr[1] + 1.0)
    y = (pr[...] + hr[...]) * dinv + br[...]
    y_out[...] = y
    if stats:
        i = pl.program_id(0)

        @pl.when(i == 0)
        def _():
            s_out[...] = jnp.zeros_like(s_out)
            q_out[...] = jnp.zeros_like(q_out)

        s_out[...] += jnp.sum(y, axis=0, keepdims=True)
        q_out[...] += jnp.sum(y * y, axis=0, keepdims=True)


def _a0_call(xt, W, degp):
    return pl.pallas_call(
        _a0_body,
        grid=(GRID,),
        in_specs=[
            pl.BlockSpec((BLK, HH), lambda i: (i, 0)),
            pl.BlockSpec((HH, HH), lambda i: (0, 0)),
            pl.BlockSpec((NC, BLK, 1), lambda i: (0, i, 0)),
        ],
        out_specs=pl.BlockSpec((BLK, HH), lambda i: (i, 0)),
        out_shape=jax.ShapeDtypeStruct((NN, HH), jnp.float32),
    )(xt, W, degp)


def _a_call(y, s, q, g, be, W, degp):
    return pl.pallas_call(
        _a_body,
        grid=(GRID,),
        in_specs=[
            pl.BlockSpec((BLK, HH), lambda i: (i, 0)),
            pl.BlockSpec((1, HH), lambda i: (0, 0)),
            pl.BlockSpec((1, HH), lambda i: (0, 0)),
            pl.BlockSpec((1, HH), lambda i: (0, 0)),
            pl.BlockSpec((1, HH), lambda i: (0, 0)),
            pl.BlockSpec((HH, HH), lambda i: (0, 0)),
            pl.BlockSpec((NC, BLK, 1), lambda i: (0, i, 0)),
        ],
        out_specs=pl.BlockSpec((BLK, HH), lambda i: (i, 0)),
        out_shape=jax.ShapeDtypeStruct((NN, HH), jnp.float32),
    )(y, s, q, g, be, W, degp)


def _b_call(parts, h, b, degp, stats):
    outs = pl.pallas_call(
        functools.partial(_b_body, stats=stats),
        grid=(GRID,),
        in_specs=[
            pl.BlockSpec((BLK, HH), lambda i: (i, 0)),
            pl.BlockSpec((BLK, HH), lambda i: (i, 0)),
            pl.BlockSpec((1, HH), lambda i: (0, 0)),
            pl.BlockSpec((NC, BLK, 1), lambda i: (0, i, 0)),
        ],
        out_specs=[
            pl.BlockSpec((BLK, HH), lambda i: (i, 0)),
            pl.BlockSpec((1, HH), lambda i: (0, 0)),
            pl.BlockSpec((1, HH), lambda i: (0, 0)),
        ],
        out_shape=[
            jax.ShapeDtypeStruct((NN, HH), jnp.float32),
            jax.ShapeDtypeStruct((1, HH), jnp.float32),
            jax.ShapeDtypeStruct((1, HH), jnp.float32),
        ],
    )(parts, h, b, degp)
    return outs


def kernel(x, edge_index, W0, b0, W1, b1, W2, b2, g0, be0, g1, be1):
    n = x.shape[1]
    xt = jnp.transpose(x, (1, 0, 2)).reshape(n, -1)
    padn = EPW - EE // NW
    src = edge_index[0].reshape(NW, EE // NW)
    dst = edge_index[1].reshape(NW, EE // NW)
    src = jnp.concatenate(
        [src, jnp.zeros((NW, padn), jnp.int32)], axis=1)
    dst = jnp.concatenate(
        [dst, jnp.full((NW, padn), NPAD - 1, jnp.int32)], axis=1)
    packed = ((src << 14) | dst).reshape(NW, NCHUNK, CH)
    zeros1 = jnp.zeros((NPAD,), jnp.float32)
    zeros2 = jnp.zeros((NPAD, HH), jnp.float32)

    degp, cnt = _cnt_call(packed, zeros1)
    degp = degp[:, :, None]
    cntT = cnt.reshape(NW, 32, 16).transpose(1, 0, 2).reshape(32, 512)
    baseT, blb, nchb = _pref_call(cntT)
    base3 = baseT.reshape(32, NW, 16).transpose(1, 0, 2).reshape(NW, 512)
    parted = _part_call(packed, base3)
    parted2 = parted.reshape(PR, CH)

    b0r = b0.reshape(1, HH)
    b1r = b1.reshape(1, HH)
    b2r = b2.reshape(1, HH)

    h0 = _a0_call(xt, W0, degp)
    p0 = _apply_call(h0, parted2, zeros2, blb, nchb)
    y0, s0, q0 = _b_call(p0, h0, b0r, degp, stats=True)

    h1 = _a_call(y0, s0, q0, g0.reshape(1, HH), be0.reshape(1, HH), W1, degp)
    p1 = _apply_call(h1, parted2, zeros2, blb, nchb)
    y1, s1, q1 = _b_call(p1, h1, b1r, degp, stats=True)

    h2 = _a_call(y1, s1, q1, g1.reshape(1, HH), be1.reshape(1, HH), W2, degp)
    p2 = _apply_call(h2, parted2, zeros2, blb, nchb)
    y2, _, _ = _b_call(p2, h2, b2r, degp, stats=False)

    return jnp.transpose(y2, (1, 0))[:, :, None]


# speedup vs baseline: 2.2858x; 2.2858x over previous
import functools

import jax
import jax.numpy as jnp
from jax import lax
from jax.experimental import pallas as pl
from jax.experimental.pallas import tpu as pltpu
from jax.experimental.pallas import tpu_sc as plsc

NN = 10000
EE = 320000
HH = 128
NC = 2
NS = 16
NW = NC * NS
NPAD = 10240
EPW = 10240
CH = 64
CHH = CH // 2
NCHUNK = EPW // CH
RPS = NPAD // NS
BLK = 1000
GRID = NN // BLK

_mesh = plsc.VectorSubcoreMesh(core_axis_name="c", subcore_axis_name="s")


def _deg_body(dsts, zeros1, out, degsh, dstv, onesv):
    c = lax.axis_index("c")
    s = lax.axis_index("s")
    wid = c * NS + s
    for j in range(CH // 16):
        onesv[pl.ds(16 * j, 16)] = jnp.ones((16,), jnp.float32)
    pltpu.sync_copy(zeros1.at[pl.ds(s * RPS, RPS)], degsh.at[pl.ds(s * RPS, RPS)])
    pltpu.sync_copy(dsts.at[wid], dstv)
    plsc.subcore_barrier()

    def chunk(i, carry):
        pltpu.sync_copy(onesv, degsh.at[dstv.at[i]], add=True)
        return carry

    lax.fori_loop(0, NCHUNK, chunk, 0)
    plsc.subcore_barrier()
    pltpu.sync_copy(degsh.at[pl.ds(s * RPS, RPS)], out.at[c, pl.ds(s * RPS, RPS)])


def _deg_call(dst, zeros1):
    kfn = pl.kernel(
        _deg_body,
        out_type=jax.ShapeDtypeStruct((NC, NPAD), jnp.float32),
        mesh=_mesh,
        scratch_types=[
            pltpu.VMEM_SHARED((NPAD,), jnp.float32),
            pltpu.VMEM((NCHUNK, CH), jnp.int32),
            pltpu.VMEM((CH,), jnp.float32),
        ],
    )
    return kfn(dst, zeros1)


def _edge_body(table, packed, zeros2, out, accsh, pk, s0v, d0a, d0b,
               s1v, d1a, d1b, rows0, rows1,
               gsem0, gsem1, sa0, sb0, sa1, sb1):
    c = lax.axis_index("c")
    s = lax.axis_index("s")
    wid = c * NS + s
    pltpu.sync_copy(zeros2.at[pl.ds(s * RPS, RPS)], accsh.at[pl.ds(s * RPS, RPS)])
    pltpu.sync_copy(packed.at[wid], pk)
    plsc.subcore_barrier()

    def unpack(i, sv, da, db):
        for j in range(CH // 16):
            p = pk[i, pl.ds(16 * j, 16)]
            sv[pl.ds(16 * j, 16)] = lax.shift_right_logical(p, 14)
            d = lax.bitwise_and(p, 16383)
            if j < CHH // 16:
                da[pl.ds(16 * j, 16)] = d
            else:
                db[pl.ds(16 * j - CHH, 16)] = d

    def scat(rows, da, db, sa, sb):
        pltpu.async_copy(rows.at[pl.ds(0, CHH)], accsh.at[da], sa, add=True)
        pltpu.async_copy(rows.at[pl.ds(CHH, CHH)], accsh.at[db], sb, add=True)

    def scat_wait(rows, da, db, sa, sb):
        pltpu.make_async_copy(rows.at[pl.ds(0, CHH)], accsh.at[da], sa).wait()
        pltpu.make_async_copy(rows.at[pl.ds(CHH, CHH)], accsh.at[db], sb).wait()

    unpack(0, s0v, d0a, d0b)
    pltpu.async_copy(table.at[s0v], rows0, gsem0)
    unpack(1, s1v, d1a, d1b)
    pltpu.async_copy(table.at[s1v], rows1, gsem1)

    def body(k, carry):
        i0 = 2 * k
        pltpu.make_async_copy(table.at[s0v], rows0, gsem0).wait()
        scat(rows0, d0a, d0b, sa0, sb0)
        pltpu.make_async_copy(table.at[s1v], rows1, gsem1).wait()
        scat(rows1, d1a, d1b, sa1, sb1)
        scat_wait(rows0, d0a, d0b, sa0, sb0)
        unpack(lax.rem(i0 + 2, NCHUNK), s0v, d0a, d0b)
        pltpu.async_copy(table.at[s0v], rows0, gsem0)
        scat_wait(rows1, d1a, d1b, sa1, sb1)
        unpack(lax.rem(i0 + 3, NCHUNK), s1v, d1a, d1b)
        pltpu.async_copy(table.at[s1v], rows1, gsem1)
        return carry

    lax.fori_loop(0, NCHUNK // 2, body, 0)
    pltpu.make_async_copy(table.at[s0v], rows0, gsem0).wait()
    pltpu.make_async_copy(table.at[s1v], rows1, gsem1).wait()
    plsc.subcore_barrier()
    pltpu.sync_copy(accsh.at[pl.ds(s * RPS, RPS)], out.at[c, pl.ds(s * RPS, RPS)])


def _edge_call(table, packed, zeros2):
    kfn = pl.kernel(
        _edge_body,
        out_type=jax.ShapeDtypeStruct((NC, NPAD, HH), jnp.float32),
        mesh=_mesh,
        scratch_types=[
            pltpu.VMEM_SHARED((NPAD, HH), jnp.float32),
            pltpu.VMEM((NCHUNK, CH), jnp.int32),
            pltpu.VMEM((CH,), jnp.int32),
            pltpu.VMEM((CHH,), jnp.int32),
            pltpu.VMEM((CHH,), jnp.int32),
            pltpu.VMEM((CH,), jnp.int32),
            pltpu.VMEM((CHH,), jnp.int32),
            pltpu.VMEM((CHH,), jnp.int32),
            pltpu.VMEM((CH, HH), jnp.float32),
            pltpu.VMEM((CH, HH), jnp.float32),
            pltpu.SemaphoreType.DMA,
            pltpu.SemaphoreType.DMA,
            pltpu.SemaphoreType.DMA,
            pltpu.SemaphoreType.DMA,
            pltpu.SemaphoreType.DMA,
            pltpu.SemaphoreType.DMA,
        ],
    )
    return kfn(table, packed, zeros2)


def _a0_body(xr, wr, degr, out):
    dinv = lax.rsqrt(degr[0] + degr[1] + 1.0)
    out[...] = jnp.dot(xr[...], wr[...],
                       preferred_element_type=jnp.float32) * dinv


def _a_body(yr, sr, qr, gr, br, wr, degr, out):
    dinv = lax.rsqrt(degr[0] + degr[1] + 1.0)
    mu = sr[...] * (1.0 / NN)
    var = qr[...] * (1.0 / NN) - mu * mu
    a = gr[...] * (yr[...] - mu) * lax.rsqrt(var + 1e-5) + br[...]
    a = jnp.maximum(a, 0.0)
    out[...] = jnp.dot(a, wr[...], preferred_element_type=jnp.float32) * dinv


def _b_body(pr, hr, br, degr, y_out, s_out, q_out, *, stats):
    dinv = lax.rsqrt(degr[0] + degr[1] + 1.0)
    y = (pr[0] + pr[1] + hr[...]) * dinv + br[...]
    y_out[...] = y
    if stats:
        i = pl.program_id(0)

        @pl.when(i == 0)
        def _():
            s_out[...] = jnp.zeros_like(s_out)
            q_out[...] = jnp.zeros_like(q_out)

        s_out[...] += jnp.sum(y, axis=0, keepdims=True)
        q_out[...] += jnp.sum(y * y, axis=0, keepdims=True)


def _a0_call(xt, W, degp):
    return pl.pallas_call(
        _a0_body,
        grid=(GRID,),
        in_specs=[
            pl.BlockSpec((BLK, HH), lambda i: (i, 0)),
            pl.BlockSpec((HH, HH), lambda i: (0, 0)),
            pl.BlockSpec((NC, BLK, 1), lambda i: (0, i, 0)),
        ],
        out_specs=pl.BlockSpec((BLK, HH), lambda i: (i, 0)),
        out_shape=jax.ShapeDtypeStruct((NN, HH), jnp.float32),
    )(xt, W, degp)


def _a_call(y, s, q, g, be, W, degp):
    return pl.pallas_call(
        _a_body,
        grid=(GRID,),
        in_specs=[
            pl.BlockSpec((BLK, HH), lambda i: (i, 0)),
            pl.BlockSpec((1, HH), lambda i: (0, 0)),
            pl.BlockSpec((1, HH), lambda i: (0, 0)),
            pl.BlockSpec((1, HH), lambda i: (0, 0)),
            pl.BlockSpec((1, HH), lambda i: (0, 0)),
            pl.BlockSpec((HH, HH), lambda i: (0, 0)),
            pl.BlockSpec((NC, BLK, 1), lambda i: (0, i, 0)),
        ],
        out_specs=pl.BlockSpec((BLK, HH), lambda i: (i, 0)),
        out_shape=jax.ShapeDtypeStruct((NN, HH), jnp.float32),
    )(y, s, q, g, be, W, degp)


def _b_call(parts, h, b, degp, stats):
    outs = pl.pallas_call(
        functools.partial(_b_body, stats=stats),
        grid=(GRID,),
        in_specs=[
            pl.BlockSpec((NC, BLK, HH), lambda i: (0, i, 0)),
            pl.BlockSpec((BLK, HH), lambda i: (i, 0)),
            pl.BlockSpec((1, HH), lambda i: (0, 0)),
            pl.BlockSpec((NC, BLK, 1), lambda i: (0, i, 0)),
        ],
        out_specs=[
            pl.BlockSpec((BLK, HH), lambda i: (i, 0)),
            pl.BlockSpec((1, HH), lambda i: (0, 0)),
            pl.BlockSpec((1, HH), lambda i: (0, 0)),
        ],
        out_shape=[
            jax.ShapeDtypeStruct((NN, HH), jnp.float32),
            jax.ShapeDtypeStruct((1, HH), jnp.float32),
            jax.ShapeDtypeStruct((1, HH), jnp.float32),
        ],
    )(parts, h, b, degp)
    return outs


def kernel(x, edge_index, W0, b0, W1, b1, W2, b2, g0, be0, g1, be1):
    n = x.shape[1]
    xt = jnp.transpose(x, (1, 0, 2)).reshape(n, -1)
    padn = EPW - EE // NW
    src = edge_index[0].reshape(NW, EE // NW)
    dst = edge_index[1].reshape(NW, EE // NW)
    src = jnp.concatenate(
        [src, jnp.zeros((NW, padn), jnp.int32)], axis=1)
    dst = jnp.concatenate(
        [dst, jnp.full((NW, padn), NPAD - 1, jnp.int32)], axis=1)
    packed = ((src << 14) | dst).reshape(NW, NCHUNK, CH)
    dst = dst.reshape(NW, NCHUNK, CH)
    zeros1 = jnp.zeros((NPAD,), jnp.float32)
    zeros2 = jnp.zeros((NPAD, HH), jnp.float32)

    degp = _deg_call(dst, zeros1)
    degp = degp[:, :, None]

    b0r = b0.reshape(1, HH)
    b1r = b1.reshape(1, HH)
    b2r = b2.reshape(1, HH)

    h0 = _a0_call(xt, W0, degp)
    p0 = _edge_call(h0, packed, zeros2)
    y0, s0, q0 = _b_call(p0, h0, b0r, degp, stats=True)

    h1 = _a_call(y0, s0, q0, g0.reshape(1, HH), be0.reshape(1, HH), W1, degp)
    p1 = _edge_call(h1, packed, zeros2)
    y1, s1, q1 = _b_call(p1, h1, b1r, degp, stats=True)

    h2 = _a_call(y1, s1, q1, g1.reshape(1, HH), be1.reshape(1, HH), W2, degp)
    p2 = _edge_call(h2, packed, zeros2)
    y2, _, _ = _b_call(p2, h2, b2r, degp, stats=False)

    return jnp.transpose(y2, (1, 0))[:, :, None]
